# Initial kernel scaffold; baseline (speedup 1.0000x reference)
import jax
import jax.numpy as jnp
from jax.experimental import pallas as pl


def kernel(z, embedding):
    raise NotImplementedError("write your pallas kernel here")


# speedup vs baseline: 1.0611x; 1.0611x over previous
import functools

import jax
import jax.numpy as jnp
from jax import lax
from jax.experimental import pallas as pl
from jax.experimental.pallas import tpu as pltpu
from jax.experimental.pallas import tpu_sc as plsc

VOCAB = 8192
ZC = 32
BETA = 0.25
N_TOK = 8 * 32 * 32

BLK = 256
GRID = N_TOK // BLK

NC = 2
NS = 16
NW = NC * NS
B_PER_W = N_TOK // NW
CH = 128
N_CH = B_PER_W // CH


CHUNK = 2048


def _argmin_body(zf_ref, embt_ref, idx_ref, zn_ref):
    z = zf_ref[...]
    zn = z / jnp.maximum(jnp.sqrt(jnp.sum(z * z, axis=1, keepdims=True)), 1e-12)
    zn_ref[...] = zn
    et = embt_ref[...]
    en_t = et / jnp.maximum(
        jnp.sqrt(jnp.sum(et * et, axis=0, keepdims=True)), 1e-12)
    zz = jnp.sum(zn * zn, axis=1, keepdims=True)
    ee = jnp.sum(en_t * en_t, axis=0, keepdims=True)
    mm = lax.dot_general(zn, en_t, (((1,), (0,)), ((), ())))
    d = (zz + ee) - 2.0 * mm
    best = jnp.full((BLK, 1), jnp.inf, jnp.float32)
    bidx = jnp.zeros((BLK, 1), jnp.int32)
    for c in range(VOCAB // CHUNK):
        dc = d[:, c * CHUNK:(c + 1) * CHUNK]
        cmin = jnp.min(dc, axis=1, keepdims=True)
        cols = c * CHUNK + lax.broadcasted_iota(jnp.int32, dc.shape, 1)
        cidx = jnp.min(jnp.where(dc == cmin, cols, jnp.int32(2**30)),
                       axis=1, keepdims=True)
        repl = (cmin < best) | ((cmin == best) & (cidx < bidx))
        bidx = jnp.where(repl, cidx, bidx)
        best = jnp.where(repl, cmin, best)
        best = best.astype(jnp.bfloat16).astype(jnp.float32)
    idx_ref[...] = bidx.reshape(1, 1, BLK)


def _tc_argmin(zf, embt):
    return pl.pallas_call(
        _argmin_body,
        grid=(GRID,),
        in_specs=[
            pl.BlockSpec((BLK, ZC), lambda i: (i, 0)),
            pl.BlockSpec((ZC, VOCAB), lambda i: (0, 0)),
        ],
        out_specs=[
            pl.BlockSpec((1, 1, BLK), lambda i: (i, 0, 0)),
            pl.BlockSpec((BLK, ZC), lambda i: (i, 0)),
        ],
        out_shape=[
            jax.ShapeDtypeStruct((GRID, 1, BLK), jnp.int32),
            jax.ShapeDtypeStruct((N_TOK, ZC), jnp.float32),
        ],
    )(zf, embt)


def _sc_body(idx_hbm, emb_hbm, zq_hbm, counts_hbm,
             idx_v, rows_v, ones_v, zeros_v, counts_sh, sem):
    c = lax.axis_index("c")
    s = lax.axis_index("s")
    wid = s * NC + c
    base = wid * B_PER_W

    for j in range(N_CH):
        pltpu.sync_copy(idx_hbm.at[pl.ds(base + j * CH, CH)], idx_v.at[j])

    zseg = VOCAB // NS
    for i in range(zseg // 16):
        zeros_v[pl.ds(i * 16, 16)] = jnp.zeros((16,), jnp.float32)
    pltpu.sync_copy(zeros_v, counts_sh.at[pl.ds(s * zseg, zseg)])
    for i in range(CH // 16):
        ones_v[pl.ds(i * 16, 16)] = jnp.full((16,), 1.0, jnp.float32)

    for j in range(N_CH):
        pltpu.async_copy(emb_hbm.at[idx_v.at[j]], rows_v.at[j], sem).wait()
        pltpu.sync_copy(rows_v.at[j], zq_hbm.at[pl.ds(base + j * CH, CH)])

    plsc.subcore_barrier()
    for j in range(N_CH):
        pltpu.sync_copy(ones_v, counts_sh.at[idx_v.at[j]], add=True)
    plsc.subcore_barrier()

    @pl.when(s == 0)
    def _():
        pltpu.sync_copy(counts_sh, counts_hbm.at[c])


def _sc_gather_count(idx, emb):
    mesh = plsc.VectorSubcoreMesh(core_axis_name="c", subcore_axis_name="s")
    fn = functools.partial(
        pl.kernel,
        out_type=(
            jax.ShapeDtypeStruct((N_TOK, ZC), jnp.float32),
            jax.ShapeDtypeStruct((NC, VOCAB), jnp.float32),
        ),
        mesh=mesh,
        scratch_types=[
            pltpu.VMEM((N_CH, CH), jnp.int32),
            pltpu.VMEM((N_CH, CH, ZC), jnp.float32),
            pltpu.VMEM((CH,), jnp.float32),
            pltpu.VMEM((VOCAB // NS,), jnp.float32),
            pltpu.VMEM_SHARED((VOCAB,), jnp.float32),
            pltpu.SemaphoreType.DMA,
        ],
        compiler_params=pltpu.CompilerParams(use_tc_tiling_on_sc=False),
    )(_sc_body)
    return fn(idx, emb)


def _final_body(zq_ref, zn_ref, cnt_ref, out_ref, usage_ref, vq_ref, com_ref):
    q = zq_ref[...]
    qn = q / jnp.maximum(jnp.sqrt(jnp.sum(q * q, axis=1, keepdims=True)), 1e-12)
    zn = zn_ref[...]
    out_ref[...] = zn + (qn - zn)
    diff = qn - zn
    m = jnp.mean(diff * diff)
    vq_ref[...] = m.reshape(1, 1)
    com_ref[...] = (BETA * m).reshape(1, 1)
    cnt = cnt_ref[...]
    tot = cnt[0:1, :] + cnt[1:2, :]
    margin = 1.0 * (float(N_TOK * ZC) / ZC) / VOCAB * 0.08
    used = (tot >= margin).astype(jnp.float32)
    usage_ref[...] = (jnp.mean(used) * 100.0).reshape(1, 1)


def _tc_final(zq_raw, zn, counts):
    return pl.pallas_call(
        _final_body,
        out_shape=[
            jax.ShapeDtypeStruct((N_TOK, ZC), jnp.float32),
            jax.ShapeDtypeStruct((1, 1), jnp.float32),
            jax.ShapeDtypeStruct((1, 1), jnp.float32),
            jax.ShapeDtypeStruct((1, 1), jnp.float32),
        ],
    )(zq_raw, zn, counts)


def kernel(z, embedding):
    b, ch, h, w = z.shape
    zf = jnp.transpose(z, (0, 2, 3, 1)).reshape(-1, ZC)
    embt = embedding.T
    idx3, zn = _tc_argmin(zf, embt)
    idx = idx3.reshape(N_TOK)
    zq_raw, counts = _sc_gather_count(idx, embedding)
    zqf, usage, vq, commit = _tc_final(zq_raw, zn, counts)
    z_q = jnp.transpose(zqf.reshape(b, h, w, ch), (0, 3, 1, 2))
    return (z_q, usage.reshape(()), vq.reshape(()), commit.reshape(()))
